# baseline (speedup 1.0000x reference)
import functools

import jax
import jax.numpy as jnp
import numpy as np
from jax import lax
from jax.experimental import pallas as pl
from jax.experimental.pallas import tpu as pltpu
from jax.experimental.pallas import tpu_sc as plsc


def _disabled_tofs(tof_count, min_c, max_c, neighbor_p, seed=0):
    rng = np.random.default_rng(seed)
    count = int(rng.integers(min_c, max_c + 1))
    tof_list = rng.permutation(tof_count)
    first = int(rng.integers(1, tof_count))
    disabled = [first]
    tof_list = tof_list[tof_list != first]
    for _ in range(count - 1):
        r = float(rng.random())
        if r < neighbor_p:
            if r < neighbor_p / 2.0:
                offsets = (1, -1)
            else:
                offsets = (tof_count // 2, -(tof_count // 2))
            appended = False
            for d in list(disabled):
                for off in offsets:
                    cand = d + off
                    if cand in tof_list:
                        tof_list = tof_list[tof_list != cand]
                        disabled.append(int(cand))
                        appended = True
                        break
                if appended:
                    break
            if not appended:
                new = int(tof_list[0])
                tof_list = tof_list[tof_list != new]
                disabled.append(new)
        else:
            new = int(tof_list[0])
            tof_list = tof_list[tof_list != new]
            disabled.append(new)
    return sorted(int(x) for x in disabled)


_ROWS, _COLS = 16384, 2048
_NW = 32
_RPW = _ROWS // _NW
_W = 128
_CHUNK = 256


@functools.cache
def _build(tof_count):
    disabled = _disabled_tofs(tof_count, 1, 3, 0.5)
    windows = sorted({(c // _W) * _W for c in disabled})
    nwin = len(windows)
    groups = {w: sorted({((c - w) // 16) * 16 for c in disabled
                         if (c // _W) * _W == w}) for w in windows}
    lanes = {w: {g: [c - w - g for c in disabled
                     if (c // _W) * _W == w and ((c - w) // 16) * 16 == g]
                 for g in groups[w]} for w in windows}
    nchunk = _RPW // _CHUNK
    mesh = plsc.VectorSubcoreMesh(core_axis_name="c", subcore_axis_name="s")

    @functools.partial(
        pl.kernel,
        mesh=mesh,
        out_type=jax.ShapeDtypeStruct((_ROWS, _COLS), jnp.float32),
        scratch_types=(
            [pltpu.VMEM((_CHUNK, _W), jnp.float32) for _ in range(nwin)]
            + [pltpu.SemaphoreType.DMA, pltpu.SemaphoreType.DMA]
        ),
    )
    def k(img_hbm, out_hbm, *rest):
        sbufs = rest[:nwin]
        bulk_sem, stripe_sem = rest[nwin], rest[nwin + 1]
        wid = lax.axis_index("s") * 2 + lax.axis_index("c")
        base = wid * _RPW
        rows = pl.ds(base, _RPW)
        iota = lax.iota(jnp.int32, 16)

        bulk = pltpu.make_async_copy(
            img_hbm.at[rows, :], out_hbm.at[rows, :], bulk_sem)
        bulk.start()

        def stripe_in(ci, w, sbuf):
            r = pl.ds(base + ci * _CHUNK, _CHUNK)
            return pltpu.make_async_copy(
                img_hbm.at[r, pl.ds(w, _W)], sbuf, stripe_sem)

        def stripe_out(ci, w, sbuf):
            r = pl.ds(base + ci * _CHUNK, _CHUNK)
            return pltpu.make_async_copy(
                sbuf, out_hbm.at[r, pl.ds(w, _W)], stripe_sem)

        for ci in range(nchunk):
            for w, sbuf in zip(windows, sbufs):
                stripe_in(ci, w, sbuf).start()
            for w, sbuf in zip(windows, sbufs):
                stripe_in(ci, w, sbuf).wait()

            def fix(r, carry):
                for w, sbuf in zip(windows, sbufs):
                    for g in groups[w]:
                        v = sbuf[r, pl.ds(g, 16)]
                        keep = jnp.ones((16,), jnp.float32)
                        for lane in lanes[w][g]:
                            keep = jnp.where(iota == lane, 0.0, keep)
                        sbuf[r, pl.ds(g, 16)] = v * keep
                return carry

            lax.fori_loop(0, _CHUNK, fix, 0)
            if ci == 0:
                bulk.wait()
            for w, sbuf in zip(windows, sbufs):
                stripe_out(ci, w, sbuf).start()
            for w, sbuf in zip(windows, sbufs):
                stripe_out(ci, w, sbuf).wait()

    return k


def kernel(img):
    return _build(img.shape[-1])(img)


# speedup vs baseline: 39.0097x; 39.0097x over previous
import functools

import jax
import jax.numpy as jnp
import numpy as np
from jax import lax
from jax.experimental import pallas as pl
from jax.experimental.pallas import tpu as pltpu
from jax.experimental.pallas import tpu_sc as plsc


def _disabled_tofs(tof_count, min_c, max_c, neighbor_p, seed=0):
    rng = np.random.default_rng(seed)
    count = int(rng.integers(min_c, max_c + 1))
    tof_list = rng.permutation(tof_count)
    first = int(rng.integers(1, tof_count))
    disabled = [first]
    tof_list = tof_list[tof_list != first]
    for _ in range(count - 1):
        r = float(rng.random())
        if r < neighbor_p:
            if r < neighbor_p / 2.0:
                offsets = (1, -1)
            else:
                offsets = (tof_count // 2, -(tof_count // 2))
            appended = False
            for d in list(disabled):
                for off in offsets:
                    cand = d + off
                    if cand in tof_list:
                        tof_list = tof_list[tof_list != cand]
                        disabled.append(int(cand))
                        appended = True
                        break
                if appended:
                    break
            if not appended:
                new = int(tof_list[0])
                tof_list = tof_list[tof_list != new]
                disabled.append(new)
        else:
            new = int(tof_list[0])
            tof_list = tof_list[tof_list != new]
            disabled.append(new)
    return sorted(int(x) for x in disabled)


_ROWS, _COLS = 16384, 2048
_NW = 32
_NS = 16
_RPW = _ROWS // _NW
_CH = 16
_N = _RPW // _CH


@functools.cache
def _build(tof_count):
    disabled = _disabled_tofs(tof_count, 1, 3, 0.5)
    windows = sorted({(c // 128) * 128 for c in disabled})
    groups = {w: sorted({((c - w) // 16) * 16 for c in disabled
                         if (c // 128) * 128 == w}) for w in windows}
    lanes = {w: {g: [c - w - g for c in disabled
                     if (c // 128) * 128 == w and ((c - w) // 16) * 16 == g]
                 for g in groups[w]} for w in windows}
    nwin = len(windows)
    mesh = plsc.VectorSubcoreMesh(core_axis_name="c", subcore_axis_name="s")

    @functools.partial(
        pl.kernel,
        mesh=mesh,
        out_type=jax.ShapeDtypeStruct((_ROWS, _COLS), jnp.float32),
        scratch_types=(
            [pltpu.VMEM_SHARED((_NS, 2, _CH, _COLS), jnp.float32)]
            + [pltpu.VMEM((_CH, 128), jnp.float32) for _ in range(nwin)]
            + [pltpu.SemaphoreType.DMA for _ in range(4)]
        ),
    )
    def k(img_hbm, out_hbm, spm, *rest):
        fbufs = rest[:nwin]
        isems = rest[nwin:nwin + 2]
        osems = rest[nwin + 2:nwin + 4]
        sid = lax.axis_index("s")
        wid = sid * 2 + lax.axis_index("c")
        iota = lax.iota(jnp.int32, 16)

        def in_cp(i, b):
            r = pl.ds((i * _NW + wid) * _CH, _CH)
            return pltpu.make_async_copy(
                img_hbm.at[r, :], spm.at[sid, b], isems[b])

        def out_cp(i, b):
            r = pl.ds((i * _NW + wid) * _CH, _CH)
            return pltpu.make_async_copy(
                spm.at[sid, b], out_hbm.at[r, :], osems[b])

        in_cp(0, 0).start()

        def body(g, carry):
            for b in range(2):
                i = g * 2 + b
                j = i + 1
                bj = 1 - b

                @pl.when(j < _N)
                def _():
                    @pl.when(j >= 2)
                    def _():
                        out_cp(j - 2, bj).wait()
                    in_cp(j, bj).start()

                in_cp(i, b).wait()
                for w, fbuf in zip(windows, fbufs):
                    pltpu.sync_copy(spm.at[sid, b, :, pl.ds(w, 128)], fbuf)

                def fix(r, carry2):
                    for w, fbuf in zip(windows, fbufs):
                        for g in groups[w]:
                            v = fbuf[r, pl.ds(g, 16)]
                            keep = jnp.ones((16,), jnp.float32)
                            for lane in lanes[w][g]:
                                keep = jnp.where(iota == lane, 0.0, keep)
                            fbuf[r, pl.ds(g, 16)] = v * keep
                    return carry2

                lax.fori_loop(0, _CH, fix, 0)
                for w, fbuf in zip(windows, fbufs):
                    pltpu.sync_copy(fbuf, spm.at[sid, b, :, pl.ds(w, 128)])
                out_cp(i, b).start()
            return carry

        lax.fori_loop(0, _N // 2, body, 0)
        for b in range(2):
            out_cp(_N - 2 + b, b).wait()

    return k


def kernel(img):
    return _build(img.shape[-1])(img)
